# baseline (speedup 1.0000x reference)
import jax
import jax.numpy as jnp
from jax.experimental import pallas as pl
from jax.experimental.pallas import tpu as pltpu
from jax.experimental.pallas import tpu_sc as plsc

B, C, H, W = 8, 96, 224, 224
N = B * H * W
EPS = 1e-4
CB = 8
NCG = C // CB
NGB = 3
NBUF = 4


def _fused_body(g0_ref, g1_ref, x_hbm, o_hbm,
                inb, outb, insem, outsem):
    def in_copy(g, b, buf):
        return pltpu.make_async_copy(
            x_hbm.at[b, pl.ds(g * CB, CB)], inb.at[buf, b], insem.at[buf, b])

    def out_copy(g, b, oslot):
        return pltpu.make_async_copy(
            outb.at[oslot], o_hbm.at[b, pl.ds(g * CB, CB)], outsem.at[oslot])

    for g in range(NGB):
        for b in range(B):
            in_copy(g, b, g).start()

    for g in range(NCG):
        buf = g % NGB
        for b in range(B):
            in_copy(g, b, buf).wait()

        s1 = jnp.zeros((CB,), jnp.float32)
        s2 = jnp.zeros((CB,), jnp.float32)
        for b in range(B):
            xb = inb[buf, b]
            s1 = s1 + jnp.sum(xb, axis=(1, 2))
            s2 = s2 + jnp.sum(xb * xb, axis=(1, 2))
        inv_n = jnp.float32(1.0 / N)
        mean = s1 * inv_n
        var = s2 * inv_n - mean * mean
        inv = jax.lax.rsqrt(var + EPS)
        a = inv[None] * (1.0 + g0_ref[g])
        c = g1_ref[g] - mean[None] * a
        a = a[:, :, None, None]
        c = c[:, :, None, None]

        for b in range(B):
            t = g * B + b
            oslot = t % NBUF
            if t >= NBUF:
                pg, pb = divmod(t - NBUF, B)
                out_copy(pg, pb, oslot).wait()
            outb[oslot] = inb[buf, b] * a[b] + c[b]
            out_copy(g, b, oslot).start()
            if g + NGB < NCG:
                in_copy(g + NGB, b, buf).start()
    for t in range(NCG * B - NBUF, NCG * B):
        pg, pb = divmod(t, B)
        out_copy(pg, pb, t % NBUF).wait()


def _sc_gather(y2, table0, table1):
    mesh = plsc.VectorSubcoreMesh(core_axis_name="c", subcore_axis_name="s")
    cp = table0.shape[1]
    out_t = jax.ShapeDtypeStruct((B, cp), table0.dtype)

    @pl.kernel(out_type=(out_t, out_t), mesh=mesh)
    def k(t0_hbm, t1_hbm, y_hbm, o0_hbm, o1_hbm):
        def body(i_vmem, o0_vmem, o1_vmem):
            pltpu.sync_copy(t0_hbm.at[i_vmem.at[0]], o0_vmem)
            pltpu.sync_copy(t1_hbm.at[i_vmem.at[0]], o1_vmem)

        pltpu.emit_pipeline(
            body,
            grid=(1,),
            in_specs=[pl.BlockSpec((1, B), lambda i: (0, 0))],
            out_specs=[pl.BlockSpec((B, cp), lambda i: (0, 0)),
                       pl.BlockSpec((B, cp), lambda i: (0, 0))],
            core_axis_name="s",
            dimension_semantics=(pltpu.PARALLEL,),
        )(y_hbm, o0_hbm, o1_hbm)

    return k(table0, table1, y2)


def kernel(x, y, embed0, embed1):
    pad = ((0, 0), (0, 128 - C))
    e0y, e1y = _sc_gather(y.reshape(1, B),
                          jnp.pad(embed0, pad), jnp.pad(embed1, pad))
    g0 = e0y[:, :C].reshape(B, NCG, CB).transpose(1, 0, 2)
    g1 = e1y[:, :C].reshape(B, NCG, CB).transpose(1, 0, 2)

    vmem = pltpu.MemorySpace.VMEM
    return pl.pallas_call(
        _fused_body,
        in_specs=[pl.BlockSpec(memory_space=vmem),
                  pl.BlockSpec(memory_space=vmem),
                  pl.BlockSpec(memory_space=pl.ANY)],
        out_specs=pl.BlockSpec(memory_space=pl.ANY),
        out_shape=jax.ShapeDtypeStruct((B, C, H, W), jnp.float32),
        scratch_shapes=[vmem((NGB, B, CB, H, W), jnp.float32),
                        vmem((NBUF, CB, H, W), jnp.float32),
                        pltpu.SemaphoreType.DMA((NGB, B)),
                        pltpu.SemaphoreType.DMA((NBUF,))],
    )(g0, g1, x)


# speedup vs baseline: 1.0023x; 1.0023x over previous
import jax
import jax.numpy as jnp
from jax.experimental import pallas as pl
from jax.experimental.pallas import tpu as pltpu
from jax.experimental.pallas import tpu_sc as plsc

B, C, H, W = 8, 96, 224, 224
N = B * H * W
EPS = 1e-4
CB = 8
NCG = C // CB
NGB = 3
NBUF = 4


def _fused_body(g0_ref, g1_ref, x_hbm, o_hbm,
                inb, outb, insem, outsem):
    def in_copy(g, buf):
        return pltpu.make_async_copy(
            x_hbm.at[:, pl.ds(g * CB, CB)], inb.at[buf], insem.at[buf])

    def out_copy(g, b, oslot):
        return pltpu.make_async_copy(
            outb.at[oslot], o_hbm.at[b, pl.ds(g * CB, CB)], outsem.at[oslot])

    for g in range(NGB):
        in_copy(g, g).start()

    for g in range(NCG):
        buf = g % NGB
        in_copy(g, buf).wait()

        s1 = jnp.zeros((CB,), jnp.float32)
        s2 = jnp.zeros((CB,), jnp.float32)
        for b in range(B):
            xb = inb[buf, b]
            s1 = s1 + jnp.sum(xb, axis=(1, 2))
            s2 = s2 + jnp.sum(xb * xb, axis=(1, 2))
        inv_n = jnp.float32(1.0 / N)
        mean = s1 * inv_n
        var = s2 * inv_n - mean * mean
        inv = jax.lax.rsqrt(var + EPS)
        a = inv[None] * (1.0 + g0_ref[g])
        c = g1_ref[g] - mean[None] * a
        a = a[:, :, None, None]
        c = c[:, :, None, None]

        for b in range(B):
            t = g * B + b
            oslot = t % NBUF
            if t >= NBUF:
                pg, pb = divmod(t - NBUF, B)
                out_copy(pg, pb, oslot).wait()
            outb[oslot] = inb[buf, b] * a[b] + c[b]
            out_copy(g, b, oslot).start()
        if g + NGB < NCG:
            in_copy(g + NGB, buf).start()
    for t in range(NCG * B - NBUF, NCG * B):
        pg, pb = divmod(t, B)
        out_copy(pg, pb, t % NBUF).wait()


def _sc_gather(y2, table0, table1):
    mesh = plsc.VectorSubcoreMesh(core_axis_name="c", subcore_axis_name="s")
    cp = table0.shape[1]
    out_t = jax.ShapeDtypeStruct((B, cp), table0.dtype)

    @pl.kernel(out_type=(out_t, out_t), mesh=mesh)
    def k(t0_hbm, t1_hbm, y_hbm, o0_hbm, o1_hbm):
        def body(i_vmem, o0_vmem, o1_vmem):
            pltpu.sync_copy(t0_hbm.at[i_vmem.at[0]], o0_vmem)
            pltpu.sync_copy(t1_hbm.at[i_vmem.at[0]], o1_vmem)

        pltpu.emit_pipeline(
            body,
            grid=(1,),
            in_specs=[pl.BlockSpec((1, B), lambda i: (0, 0))],
            out_specs=[pl.BlockSpec((B, cp), lambda i: (0, 0)),
                       pl.BlockSpec((B, cp), lambda i: (0, 0))],
            core_axis_name="s",
            dimension_semantics=(pltpu.PARALLEL,),
        )(y_hbm, o0_hbm, o1_hbm)

    return k(table0, table1, y2)


def kernel(x, y, embed0, embed1):
    pad = ((0, 0), (0, 128 - C))
    e0y, e1y = _sc_gather(y.reshape(1, B),
                          jnp.pad(embed0, pad), jnp.pad(embed1, pad))
    g0 = e0y[:, :C].reshape(B, NCG, CB).transpose(1, 0, 2)
    g1 = e1y[:, :C].reshape(B, NCG, CB).transpose(1, 0, 2)

    vmem = pltpu.MemorySpace.VMEM
    return pl.pallas_call(
        _fused_body,
        in_specs=[pl.BlockSpec(memory_space=vmem),
                  pl.BlockSpec(memory_space=vmem),
                  pl.BlockSpec(memory_space=pl.ANY)],
        out_specs=pl.BlockSpec(memory_space=pl.ANY),
        out_shape=jax.ShapeDtypeStruct((B, C, H, W), jnp.float32),
        scratch_shapes=[vmem((NGB, B, CB, H, W), jnp.float32),
                        vmem((NBUF, CB, H, W), jnp.float32),
                        pltpu.SemaphoreType.DMA((NGB,)),
                        pltpu.SemaphoreType.DMA((NBUF,))],
    )(g0, g1, x)
